# baseline (speedup 1.0000x reference)
import jax
import jax.numpy as jnp
from jax.experimental import pallas as pl
from jax.experimental.pallas import tpu as pltpu


def _body(a_smem, h_ref, r_ref, msrc_ref, mr_ref, wrow_ref, wac_ref,
          out_ref, pp_ref, qq_ref, sums_ref):
    @pl.when(pl.program_id(0) == 0)
    def _init():
        pp_ref[...] = jnp.dot(h_ref[...], msrc_ref[...],
                              preferred_element_type=jnp.float32)
        qq_ref[...] = jnp.dot(r_ref[...], mr_ref[...],
                              preferred_element_type=jnp.float32)
        out_ref[...] = jnp.zeros_like(out_ref)
        sums_ref[...] = jnp.zeros_like(sums_ref)

    E = a_smem.shape[2]

    def step(e, carry):
        dst = a_smem[0, 0, e]
        ri = a_smem[0, 1, e]
        si = a_smem[0, 2, e]
        g = h_ref[pl.ds(si, 1), :]
        rr = r_ref[pl.ds(ri, 1), :]
        pu = pp_ref[pl.ds(si, 1), :]
        qv = qq_ref[pl.ds(ri, 1), :]
        c_col = pu[:, 2:3]
        x = pu[:, 0:2] + qv[:, 0:2] + c_col * wac_ref[:, 0:2]
        ev = jnp.exp(-jnp.where(x >= 0.0, x, 0.2 * x))
        dd = (g - rr) - (2.0 * c_col) * wrow_ref[...]
        out_ref[0, pl.ds(dst, 1), :] += ev[:, 0:1] * dd
        out_ref[1, pl.ds(dst, 1), :] += ev[:, 1:2] * dd
        sums_ref[pl.ds(dst, 1), 0:2] += ev
        return carry

    jax.lax.fori_loop(0, E, step, 0)

    @pl.when(pl.program_id(0) == pl.num_programs(0) - 1)
    def _fini():
        for i in range(2):
            out_ref[i, :, :] = out_ref[i, :, :] / sums_ref[:, i:i + 1]


def _pair_call(a_blocks, h, inputr, msrc, mr, wrow, wac):
    NB, _, EC = a_blocks.shape
    N, F = h.shape
    full = lambda *shape: pl.BlockSpec(shape, lambda i: (0,) * len(shape),
                                       memory_space=pltpu.VMEM)
    return pl.pallas_call(
        _body,
        grid=(NB,),
        out_shape=jax.ShapeDtypeStruct((2, N, F), jnp.float32),
        in_specs=[
            pl.BlockSpec((1, 3, EC), lambda i: (i, 0, 0),
                         memory_space=pltpu.SMEM),
            full(N, F),
            full(N, F),
            full(F, 8),
            full(F, 8),
            full(1, F),
            full(1, 8),
        ],
        out_specs=full(2, N, F),
        scratch_shapes=[
            pltpu.VMEM((N, 8), jnp.float32),
            pltpu.VMEM((N, 8), jnp.float32),
            pltpu.VMEM((N, 8), jnp.float32),
        ],
    )(a_blocks, h, inputr, msrc, mr, wrow, wac)


def kernel(h, inputr, A, cfdc, sup_ents, w_ori, a_src_dst):
    del cfdc, sup_ents
    N, F = h.shape
    E = A.shape[1]
    f32 = jnp.float32

    norm = jnp.sqrt(jnp.sum(w_ori * w_ori, axis=1, keepdims=True))
    w = w_ori / jnp.maximum(norm, 1e-12)
    a_src = a_src_dst[:, 0, :, 0]
    a_dst = a_src_dst[:, 1, :, 0]
    wa = (w @ a_src.T)[0]
    zpad = jnp.zeros((F, 5), f32)
    z8 = jnp.zeros((1, 8), f32)

    NB = 8
    while E % NB != 0:
        NB -= 1
    EC = E // NB
    a_blocks = A.reshape(3, NB, EC).transpose(1, 0, 2)

    outs = []
    for pair, reflect in (((0, 2), False), ((1, 3), True)):
        p, q = pair
        msrc = jnp.concatenate(
            [a_src[p:p + 1].T, a_src[q:q + 1].T, w.T, zpad], axis=1)
        mr = jnp.concatenate(
            [a_dst[p:p + 1].T, a_dst[q:q + 1].T, zpad, w.T * 0], axis=1)
        if reflect:
            wrow = w
            wac = z8.at[0, 0].set(-2.0 * wa[p]).at[0, 1].set(-2.0 * wa[q])
        else:
            wrow = w * 0.0
            wac = z8
        outs.append(_pair_call(a_blocks, h, inputr, msrc, mr, wrow, wac))

    o02, o13 = outs
    return jnp.stack([o02[0], o13[0], o02[1], o13[1]], axis=0)
